# baseline (speedup 1.0000x reference)
import functools

import jax
import jax.numpy as jnp
from jax import lax
from jax.experimental import pallas as pl
from jax.experimental.pallas import tpu as pltpu
from jax.experimental.pallas import tpu_sc as plsc

KNB = 16
LSEQ = 2048
DATT = 128
NBATCH = 4
RBLK = 1024

_NC = 2
_NS = 16
_NW = _NC * _NS
_TOTAL_ROWS = NBATCH * LSEQ * KNB
_BATCH_ROWS = LSEQ * KNB
_ROWS_PER_W = _BATCH_ROWS // _NW
_CHUNK = 128
_NCH = _ROWS_PER_W // _CHUNK


def _topk_body(b, ci_ref, cj_ref, nbr_ref, dist_ref):
    ci = ci_ref[0]
    cj = cj_ref[0]
    ad = jnp.abs(ci - cj)
    j = lax.broadcasted_iota(jnp.int32, ad.shape, 1)
    keys = (ad << 11) | j
    q = jnp.full((RBLK, 1), jnp.int32(-(2 ** 31)))
    nbr_cols = []
    dist_cols = []
    for _ in range(KNB):
        m = jnp.min(keys - q, axis=1, keepdims=True)
        mi = m + q
        q = mi + jnp.int32(-(2 ** 31) + 1)
        nbr_cols.append((mi & 0x7FF) + b * LSEQ)
        dist_cols.append((mi >> 11).astype(jnp.float32))
    nbr_ref[0] = jnp.concatenate(nbr_cols, axis=1)
    dist_ref[0] = jnp.concatenate(dist_cols, axis=1)


@functools.cache
def _make_topk_call(b):
    return pl.pallas_call(
        functools.partial(_topk_body, b),
        grid=(LSEQ // RBLK,),
        in_specs=[
            pl.BlockSpec((1, RBLK, 1), lambda i: (0, i, 0)),
            pl.BlockSpec((1, 1, LSEQ), lambda i: (0, 0, 0)),
        ],
        out_specs=[
            pl.BlockSpec((1, RBLK, KNB), lambda i: (0, i, 0)),
            pl.BlockSpec((1, RBLK, KNB), lambda i: (0, i, 0)),
        ],
        out_shape=[
            jax.ShapeDtypeStruct((1, LSEQ, KNB), jnp.int32),
            jax.ShapeDtypeStruct((1, LSEQ, KNB), jnp.float32),
        ],
    )


def _gather_body(b, table_hbm, idx_hbm, out_hbm, idx_all, rows0, rows1,
                 gsem0, gsem1, ssem0, ssem1):
    wid = lax.axis_index("s") * _NC + lax.axis_index("c")
    base = b * _BATCH_ROWS + wid * _ROWS_PER_W
    pltpu.sync_copy(idx_hbm.at[wid], idx_all)

    def gather(ch, rows, sem):
        pltpu.make_async_copy(table_hbm.at[idx_all.at[ch]], rows, sem).start()

    def wait_gather(rows, sem):
        pltpu.make_async_copy(table_hbm.at[idx_all.at[0]], rows, sem).wait()

    def store(ch, rows, sem):
        dst = out_hbm.at[pl.ds(base + ch * _CHUNK, _CHUNK)]
        pltpu.make_async_copy(rows, dst, sem).start()

    def wait_store(rows, sem):
        dst = out_hbm.at[pl.ds(base, _CHUNK)]
        pltpu.make_async_copy(rows, dst, sem).wait()

    gather(0, rows0, gsem0)
    gather(1, rows1, gsem1)

    def step(t, carry):
        a = 2 * t
        wait_gather(rows0, gsem0)
        store(a, rows0, ssem0)
        wait_gather(rows1, gsem1)
        store(a + 1, rows1, ssem1)

        @pl.when(t < _NCH // 2 - 1)
        def _():
            wait_store(rows0, ssem0)
            gather(a + 2, rows0, gsem0)
            wait_store(rows1, ssem1)
            gather(a + 3, rows1, gsem1)

        return carry

    lax.fori_loop(0, _NCH // 2, step, 0)
    wait_store(rows0, ssem0)
    wait_store(rows1, ssem1)


@functools.cache
def _make_gather_call(b):
    return pl.kernel(
        functools.partial(_gather_body, b),
        out_type=(),
        mesh=plsc.VectorSubcoreMesh(
            core_axis_name="c", subcore_axis_name="s",
            num_cores=_NC, num_subcores=_NS,
        ),
        scratch_types=[
            pltpu.VMEM((_NCH, _CHUNK), jnp.int32),
            pltpu.VMEM((_CHUNK, DATT), jnp.float32),
            pltpu.VMEM((_CHUNK, DATT), jnp.float32),
            pltpu.SemaphoreType.DMA,
            pltpu.SemaphoreType.DMA,
            pltpu.SemaphoreType.DMA,
            pltpu.SemaphoreType.DMA,
        ],
    )


@jax.jit
def kernel(first_index, attribute):
    B, L, _ = first_index.shape
    table = attribute.reshape(B * L, DATT)
    rows_ref = jax.new_ref(lax.empty((_TOTAL_ROWS, DATT), jnp.float32))
    dists = []
    for b in range(B):
        fib = lax.slice_in_dim(first_index, b, b + 1, axis=0)
        nbr, dist = _make_topk_call(b)(fib, fib.reshape(1, 1, L))
        _make_gather_call(b)(table, nbr.reshape(_NW, _NCH, _CHUNK), rows_ref)
        dists.append(dist)
    nb_attr = rows_ref[...].reshape(B, L, KNB, DATT)
    index_distance = jnp.concatenate(dists, axis=0).reshape(B, L, KNB, 1)
    return (index_distance, nb_attr)


# speedup vs baseline: 1.1877x; 1.1877x over previous
import functools

import jax
import jax.numpy as jnp
from jax import lax
from jax.experimental import pallas as pl
from jax.experimental.pallas import tpu as pltpu
from jax.experimental.pallas import tpu_sc as plsc

KNB = 16
LSEQ = 2048
DATT = 128
NBATCH = 4
RBLK = 512

_NC = 2
_NS = 16
_NW = _NC * _NS
_TOTAL_ROWS = NBATCH * LSEQ * KNB
_BATCH_ROWS = LSEQ * KNB
_ROWS_PER_W = _BATCH_ROWS // _NW
_CHUNK = 128
_NCH = _ROWS_PER_W // _CHUNK


def _topk_body(b, ci_ref, cj_ref, nbr_ref, dist_ref):
    ci = ci_ref[0]
    cj = cj_ref[0]
    ad = jnp.abs(ci - cj)
    j = lax.broadcasted_iota(jnp.int32, ad.shape, 1)
    keys = (ad << 11) | j
    q = jnp.full((RBLK, 1), jnp.int32(-(2 ** 31)))
    nbr_cols = []
    dist_cols = []
    for _ in range(KNB):
        m = jnp.min(keys - q, axis=1, keepdims=True)
        mi = m + q
        q = mi + jnp.int32(-(2 ** 31) + 1)
        nbr_cols.append((mi & 0x7FF) + b * LSEQ)
        dist_cols.append((mi >> 11).astype(jnp.float32))
    nbr_ref[0] = jnp.concatenate(nbr_cols, axis=1)
    dist_ref[0] = jnp.concatenate(dist_cols, axis=1)


@functools.cache
def _make_topk_call(b):
    return pl.pallas_call(
        functools.partial(_topk_body, b),
        grid=(LSEQ // RBLK,),
        in_specs=[
            pl.BlockSpec((1, RBLK, 1), lambda i: (0, i, 0)),
            pl.BlockSpec((1, 1, LSEQ), lambda i: (0, 0, 0)),
        ],
        out_specs=[
            pl.BlockSpec((1, RBLK, KNB), lambda i: (0, i, 0)),
            pl.BlockSpec((1, RBLK, KNB), lambda i: (0, i, 0)),
        ],
        out_shape=[
            jax.ShapeDtypeStruct((1, LSEQ, KNB), jnp.int32),
            jax.ShapeDtypeStruct((1, LSEQ, KNB), jnp.float32),
        ],
    )


def _gather_body(b, table_hbm, idx_hbm, out_hbm, idx_all, rows0, rows1,
                 gsem0, gsem1, ssem0, ssem1):
    wid = lax.axis_index("s") * _NC + lax.axis_index("c")
    base = b * _BATCH_ROWS + wid * _ROWS_PER_W
    pltpu.sync_copy(idx_hbm.at[wid], idx_all)

    def gather(ch, rows, sem):
        pltpu.make_async_copy(table_hbm.at[idx_all.at[ch]], rows, sem).start()

    def wait_gather(rows, sem):
        pltpu.make_async_copy(table_hbm.at[idx_all.at[0]], rows, sem).wait()

    def store(ch, rows, sem):
        dst = out_hbm.at[pl.ds(base + ch * _CHUNK, _CHUNK)]
        pltpu.make_async_copy(rows, dst, sem).start()

    def wait_store(rows, sem):
        dst = out_hbm.at[pl.ds(base, _CHUNK)]
        pltpu.make_async_copy(rows, dst, sem).wait()

    gather(0, rows0, gsem0)
    gather(1, rows1, gsem1)

    def step(t, carry):
        a = 2 * t
        wait_gather(rows0, gsem0)
        store(a, rows0, ssem0)
        wait_gather(rows1, gsem1)
        store(a + 1, rows1, ssem1)

        @pl.when(t < _NCH // 2 - 1)
        def _():
            wait_store(rows0, ssem0)
            gather(a + 2, rows0, gsem0)
            wait_store(rows1, ssem1)
            gather(a + 3, rows1, gsem1)

        return carry

    lax.fori_loop(0, _NCH // 2, step, 0)
    wait_store(rows0, ssem0)
    wait_store(rows1, ssem1)


@functools.cache
def _make_gather_call(b):
    return pl.kernel(
        functools.partial(_gather_body, b),
        out_type=(),
        mesh=plsc.VectorSubcoreMesh(
            core_axis_name="c", subcore_axis_name="s",
            num_cores=_NC, num_subcores=_NS,
        ),
        scratch_types=[
            pltpu.VMEM((_NCH, _CHUNK), jnp.int32),
            pltpu.VMEM((_CHUNK, DATT), jnp.float32),
            pltpu.VMEM((_CHUNK, DATT), jnp.float32),
            pltpu.SemaphoreType.DMA,
            pltpu.SemaphoreType.DMA,
            pltpu.SemaphoreType.DMA,
            pltpu.SemaphoreType.DMA,
        ],
    )


@jax.jit
def kernel(first_index, attribute):
    B, L, _ = first_index.shape
    table = attribute.reshape(B * L, DATT)
    rows_ref = jax.new_ref(lax.empty((_TOTAL_ROWS, DATT), jnp.float32))
    dists = []
    for b in range(B):
        fib = lax.slice_in_dim(first_index, b, b + 1, axis=0)
        nbr, dist = _make_topk_call(b)(fib, fib.reshape(1, 1, L))
        _make_gather_call(b)(table, nbr.reshape(_NW, _NCH, _CHUNK), rows_ref)
        dists.append(dist)
    nb_attr = rows_ref[...].reshape(B, L, KNB, DATT)
    index_distance = jnp.concatenate(dists, axis=0).reshape(B, L, KNB, 1)
    return (index_distance, nb_attr)


# speedup vs baseline: 1.2215x; 1.0285x over previous
import functools

import jax
import jax.numpy as jnp
from jax import lax
from jax.experimental import pallas as pl
from jax.experimental.pallas import tpu as pltpu
from jax.experimental.pallas import tpu_sc as plsc

KNB = 16
LSEQ = 2048
DATT = 128
NBATCH = 4
RBLK = 512

_NC = 2
_NS = 16
_NW = _NC * _NS
_TOTAL_ROWS = NBATCH * LSEQ * KNB
_BATCH_ROWS = LSEQ * KNB
_ROWS_PER_W = _BATCH_ROWS // _NW
_CHUNK = 128
_NCH = _ROWS_PER_W // _CHUNK


def _topk_body(b_base, ci_ref, cj_ref, nbr_ref, dist_ref):
    ci = ci_ref[0]
    cj = cj_ref[0]
    b = b_base + pl.program_id(0)
    ad = jnp.abs(ci - cj)
    j = lax.broadcasted_iota(jnp.int32, ad.shape, 1)
    keys = (ad << 11) | j
    q = jnp.full((RBLK, 1), jnp.int32(-(2 ** 31)))
    nbr_cols = []
    dist_cols = []
    for _ in range(KNB):
        m = jnp.min(keys - q, axis=1, keepdims=True)
        mi = m + q
        q = mi + jnp.int32(-(2 ** 31) + 1)
        nbr_cols.append((mi & 0x7FF) + b * LSEQ)
        dist_cols.append((mi >> 11).astype(jnp.float32))
    nbr_ref[0] = jnp.concatenate(nbr_cols, axis=1)
    dist_ref[0] = jnp.concatenate(dist_cols, axis=1)


@functools.cache
def _make_topk_call(b_base, nb):
    return pl.pallas_call(
        functools.partial(_topk_body, b_base),
        grid=(nb, LSEQ // RBLK),
        in_specs=[
            pl.BlockSpec((1, RBLK, 1), lambda pb, i: (pb, i, 0)),
            pl.BlockSpec((1, 1, LSEQ), lambda pb, i: (pb, 0, 0)),
        ],
        out_specs=[
            pl.BlockSpec((1, RBLK, KNB), lambda pb, i: (pb, i, 0)),
            pl.BlockSpec((1, RBLK, KNB), lambda pb, i: (pb, i, 0)),
        ],
        out_shape=[
            jax.ShapeDtypeStruct((nb, LSEQ, KNB), jnp.int32),
            jax.ShapeDtypeStruct((nb, LSEQ, KNB), jnp.float32),
        ],
    )


def _gather_body(row_base, nch, table_hbm, idx_hbm, out_hbm, idx_all,
                 rows0, rows1, gsem0, gsem1, ssem0, ssem1):
    wid = lax.axis_index("s") * _NC + lax.axis_index("c")
    base = row_base + wid * (nch * _CHUNK)
    pltpu.sync_copy(idx_hbm.at[wid], idx_all)

    def gather(ch, rows, sem):
        pltpu.make_async_copy(table_hbm.at[idx_all.at[ch]], rows, sem).start()

    def wait_gather(rows, sem):
        pltpu.make_async_copy(table_hbm.at[idx_all.at[0]], rows, sem).wait()

    def store(ch, rows, sem):
        dst = out_hbm.at[pl.ds(base + ch * _CHUNK, _CHUNK)]
        pltpu.make_async_copy(rows, dst, sem).start()

    def wait_store(rows, sem):
        dst = out_hbm.at[pl.ds(base, _CHUNK)]
        pltpu.make_async_copy(rows, dst, sem).wait()

    gather(0, rows0, gsem0)
    gather(1, rows1, gsem1)

    def step(t, carry):
        a = 2 * t
        wait_gather(rows0, gsem0)
        store(a, rows0, ssem0)
        wait_gather(rows1, gsem1)
        store(a + 1, rows1, ssem1)

        @pl.when(t < nch // 2 - 1)
        def _():
            wait_store(rows0, ssem0)
            gather(a + 2, rows0, gsem0)
            wait_store(rows1, ssem1)
            gather(a + 3, rows1, gsem1)

        return carry

    lax.fori_loop(0, nch // 2, step, 0)
    wait_store(rows0, ssem0)
    wait_store(rows1, ssem1)


@functools.cache
def _make_gather_call(row_base, nch):
    return pl.kernel(
        functools.partial(_gather_body, row_base, nch),
        out_type=(),
        mesh=plsc.VectorSubcoreMesh(
            core_axis_name="c", subcore_axis_name="s",
            num_cores=_NC, num_subcores=_NS,
        ),
        scratch_types=[
            pltpu.VMEM((nch, _CHUNK), jnp.int32),
            pltpu.VMEM((_CHUNK, DATT), jnp.float32),
            pltpu.VMEM((_CHUNK, DATT), jnp.float32),
            pltpu.SemaphoreType.DMA,
            pltpu.SemaphoreType.DMA,
            pltpu.SemaphoreType.DMA,
            pltpu.SemaphoreType.DMA,
        ],
    )


@jax.jit
def kernel(first_index, attribute):
    B, L, _ = first_index.shape
    table = attribute.reshape(B * L, DATT)
    rows_ref = jax.new_ref(lax.empty((_TOTAL_ROWS, DATT), jnp.float32))
    dists = []
    fi01 = lax.slice_in_dim(first_index, 0, 2, axis=0)
    nbr01, dist01 = _make_topk_call(0, 2)(fi01, fi01.reshape(2, 1, L))
    _make_gather_call(0, 2 * _NCH)(
        table, nbr01.reshape(_NW, 2 * _NCH, _CHUNK), rows_ref)
    dists.append(dist01)
    for b in range(2, B):
        fib = lax.slice_in_dim(first_index, b, b + 1, axis=0)
        nbr, dist = _make_topk_call(b, 1)(fib, fib.reshape(1, 1, L))
        _make_gather_call(b * _BATCH_ROWS, _NCH)(
            table, nbr.reshape(_NW, _NCH, _CHUNK), rows_ref)
        dists.append(dist)
    nb_attr = rows_ref[...].reshape(B, L, KNB, DATT)
    index_distance = jnp.concatenate(dists, axis=0).reshape(B, L, KNB, 1)
    return (index_distance, nb_attr)
